# baseline (speedup 1.0000x reference)
import jax
import jax.numpy as jnp
from jax import lax
from jax.experimental import pallas as pl
from jax.experimental.pallas import tpu as pltpu
from jax.experimental.pallas import tpu_sc as plsc

B, D, V, K1 = 1024, 128, 1000000, 512
T = 0.07
L = 16
NW = 32
B_PER_W = B // NW
C_ROWS = 128
N_CHUNKS = B_PER_W * K1 // C_ROWS
G_PER_C = C_ROWS // L
NBUF = 4
NC = D // L


def _fire(mem_hbm, idx_all, rows_v, chunk, buf, sem):
    pltpu.async_copy(
        mem_hbm.at[idx_all.at[chunk]],
        rows_v.at[pl.ds(buf * C_ROWS, C_ROWS)],
        sem,
    )


def _wait(mem_hbm, idx_all, rows_v, chunk, buf, sem):
    pltpu.make_async_copy(
        mem_hbm.at[idx_all.at[chunk]],
        rows_v.at[pl.ds(buf * C_ROWS, C_ROWS)],
        sem,
    ).wait()


def _body(x_hbm, idx_hbm, mem_hbm, out_hbm, idx_all, rows_v, out_v, xall_v,
          sem0, sem1, sem2, sem3):
    sems = (sem0, sem1, sem2, sem3)
    wid = lax.axis_index("s") * 2 + lax.axis_index("c")
    b0 = wid * B_PER_W
    iota = lax.iota(jnp.int32, L)
    inv_t = jnp.float32(1.0 / T)

    pltpu.sync_copy(idx_hbm.at[pl.ds(b0 * 4, B_PER_W * 4)], idx_all)
    pltpu.sync_copy(x_hbm.at[pl.ds(b0, B_PER_W)], xall_v)

    for c in range(NBUF - 1):
        _fire(mem_hbm, idx_all, rows_v, c, c, sems[c])

    def per_chunk(t, _):
        a = t >> 2

        for bb in range(NBUF):
            @pl.when((t & 3) == bb)
            def _(bb=bb):
                nb = (bb + NBUF - 1) & 3

                @pl.when(t < N_CHUNKS - (NBUF - 1))
                def _():
                    _fire(mem_hbm, idx_all, rows_v, t + NBUF - 1, nb,
                          sems[nb])
                _wait(mem_hbm, idx_all, rows_v, t, bb, sems[bb])

        row_base = (t & 3) * C_ROWS

        def per_group(g, _):
            row_ids = row_base + g * L + iota
            xvs = [xall_v[a, pl.ds(c * L, L)] for c in range(NC)]

            def rbody(r, carry):
                rot = carry[0]
                accs = list(carry[1:])
                for c in range(NC):
                    cidx = rot + c * L
                    col = plsc.load_gather(rows_v, [row_ids, cidx])
                    xs = xvs[c].at[rot].get(mode="promise_in_bounds")
                    accs[c] = accs[c] + col * xs
                return ((rot + 1) & (L - 1),) + tuple(accs)

            z = jnp.zeros((L,), jnp.float32)
            res = lax.fori_loop(0, L, rbody, (iota,) + (z,) * NC)
            accs = res[1:]
            s01 = accs[0] + accs[1]
            s23 = accs[2] + accs[3]
            s45 = accs[4] + accs[5]
            s67 = accs[6] + accs[7]
            out_v[pl.ds(t * C_ROWS + g * L, L)] = (
                (s01 + s23) + (s45 + s67)) * inv_t
            return 0

        lax.fori_loop(0, G_PER_C, per_group, 0)
        return 0

    lax.fori_loop(0, N_CHUNKS, per_chunk, 0)
    pltpu.sync_copy(out_v, out_hbm.at[pl.ds(b0 * K1, B_PER_W * K1)])


@jax.jit
def _run(x, idx2, memory):
    kfn = pl.kernel(
        _body,
        out_type=jax.ShapeDtypeStruct((B * K1,), jnp.float32),
        mesh=plsc.VectorSubcoreMesh(core_axis_name="c", subcore_axis_name="s"),
        compiler_params=pltpu.CompilerParams(needs_layout_passes=False),
        scratch_types=[
            pltpu.VMEM((B_PER_W * 4, 128), jnp.int32),
            pltpu.VMEM((NBUF * C_ROWS, D), jnp.float32),
            pltpu.VMEM((B_PER_W * K1,), jnp.float32),
            pltpu.VMEM((B_PER_W, D), jnp.float32),
            pltpu.SemaphoreType.DMA,
            pltpu.SemaphoreType.DMA,
            pltpu.SemaphoreType.DMA,
            pltpu.SemaphoreType.DMA,
        ],
    )
    return kfn(x, idx2, memory)


def kernel(x, y, idx, memory):
    del y
    idx2 = idx.reshape(B * 4, 128)
    return _run(x, idx2, memory).reshape(B, K1)


# speedup vs baseline: 1.1027x; 1.1027x over previous
import jax
import jax.numpy as jnp
from jax import lax
from jax.experimental import pallas as pl
from jax.experimental.pallas import tpu as pltpu
from jax.experimental.pallas import tpu_sc as plsc

B, D, V, K1 = 1024, 128, 1000000, 512
T = 0.07
L = 16
NW = 32
B_PER_W = B // NW
C_ROWS = 256
N_CHUNKS = B_PER_W * 2
G_PER_C = C_ROWS // L
NC = D // L


def _fire(mem_hbm, idx_all, rows_v, chunk, buf, sem):
    for r in range(2):
        pltpu.async_copy(
            mem_hbm.at[idx_all.at[chunk * 2 + r]],
            rows_v.at[pl.ds(buf * C_ROWS + r * 128, 128)],
            sem,
        )


def _wait(mem_hbm, idx_all, rows_v, chunk, buf, sem):
    for r in range(2):
        pltpu.make_async_copy(
            mem_hbm.at[idx_all.at[chunk * 2 + r]],
            rows_v.at[pl.ds(buf * C_ROWS + r * 128, 128)],
            sem,
        ).wait()


def _body(x_hbm, idx_hbm, mem_hbm, out_hbm, idx_all, rows_v, out_v, xall_v,
          sem0, sem1):
    wid = lax.axis_index("s") * 2 + lax.axis_index("c")
    b0 = wid * B_PER_W
    iota = lax.iota(jnp.int32, L)
    inv_t = jnp.float32(1.0 / T)

    pltpu.sync_copy(idx_hbm.at[pl.ds(b0 * 4, B_PER_W * 4)], idx_all)
    pltpu.sync_copy(x_hbm.at[pl.ds(b0, B_PER_W)], xall_v)

    _fire(mem_hbm, idx_all, rows_v, 0, 0, sem0)

    def per_chunk(t, _):
        p = t & 1
        a = t >> 1

        @pl.when(p == 0)
        def _():
            @pl.when(t < N_CHUNKS - 1)
            def _():
                _fire(mem_hbm, idx_all, rows_v, t + 1, 1, sem1)
            _wait(mem_hbm, idx_all, rows_v, t, 0, sem0)

        @pl.when(p == 1)
        def _():
            @pl.when(t < N_CHUNKS - 1)
            def _():
                _fire(mem_hbm, idx_all, rows_v, t + 1, 0, sem0)
            _wait(mem_hbm, idx_all, rows_v, t, 1, sem1)

        row_base = p * C_ROWS

        def per_gpair(g, _):
            row_ids0 = row_base + g * (2 * L) + iota
            row_ids1 = row_ids0 + L
            xvs = [xall_v[a, pl.ds(c * L, L)] for c in range(NC)]

            def rbody(r, carry):
                rot = carry[0]
                accs = list(carry[1:])
                for c in range(NC):
                    cidx = rot + c * L
                    xs = xvs[c].at[rot].get(mode="promise_in_bounds")
                    col0 = plsc.load_gather(rows_v, [row_ids0, cidx])
                    col1 = plsc.load_gather(rows_v, [row_ids1, cidx])
                    accs[c] = accs[c] + col0 * xs
                    accs[NC + c] = accs[NC + c] + col1 * xs
                return ((rot + 1) & (L - 1),) + tuple(accs)

            z = jnp.zeros((L,), jnp.float32)
            res = lax.fori_loop(0, L, rbody, (iota,) + (z,) * (2 * NC),
                                unroll=4)
            accs = res[1:]
            for h in range(2):
                aa = accs[h * NC:(h + 1) * NC]
                s01 = aa[0] + aa[1]
                s23 = aa[2] + aa[3]
                s45 = aa[4] + aa[5]
                s67 = aa[6] + aa[7]
                out_v[pl.ds(t * C_ROWS + g * 2 * L + h * L, L)] = (
                    (s01 + s23) + (s45 + s67)) * inv_t
            return 0

        lax.fori_loop(0, G_PER_C // 2, per_gpair, 0)
        return 0

    lax.fori_loop(0, N_CHUNKS, per_chunk, 0)
    pltpu.sync_copy(out_v, out_hbm.at[pl.ds(b0 * K1, B_PER_W * K1)])


@jax.jit
def _run(x, idx2, memory):
    kfn = pl.kernel(
        _body,
        out_type=jax.ShapeDtypeStruct((B * K1,), jnp.float32),
        mesh=plsc.VectorSubcoreMesh(core_axis_name="c", subcore_axis_name="s"),
        compiler_params=pltpu.CompilerParams(needs_layout_passes=False),
        scratch_types=[
            pltpu.VMEM((B_PER_W * 4, 128), jnp.int32),
            pltpu.VMEM((2 * C_ROWS, D), jnp.float32),
            pltpu.VMEM((B_PER_W * K1,), jnp.float32),
            pltpu.VMEM((B_PER_W, D), jnp.float32),
            pltpu.SemaphoreType.DMA,
            pltpu.SemaphoreType.DMA,
        ],
    )
    return kfn(x, idx2, memory)


def kernel(x, y, idx, memory):
    del y
    idx2 = idx.reshape(B * 4, 128)
    return _run(x, idx2, memory).reshape(B, K1)


# speedup vs baseline: 1.1171x; 1.0131x over previous
import jax
import jax.numpy as jnp
from jax import lax
from jax.experimental import pallas as pl
from jax.experimental.pallas import tpu as pltpu
from jax.experimental.pallas import tpu_sc as plsc

B, D, V, K1 = 1024, 128, 1000000, 512
T = 0.07
L = 16
NW = 32
B_PER_W = B // NW
C_ROWS = 256
N_CHUNKS = B_PER_W * 2
G_PER_C = C_ROWS // L
NC = D // L


def _fire(mem_hbm, idx_all, rows_v, chunk, buf, sem):
    for r in range(2):
        pltpu.async_copy(
            mem_hbm.at[idx_all.at[chunk * 2 + r]],
            rows_v.at[pl.ds(buf * C_ROWS + r * 128, 128)],
            sem,
        )


def _wait(mem_hbm, idx_all, rows_v, chunk, buf, sem):
    for r in range(2):
        pltpu.make_async_copy(
            mem_hbm.at[idx_all.at[chunk * 2 + r]],
            rows_v.at[pl.ds(buf * C_ROWS + r * 128, 128)],
            sem,
        ).wait()


def _body(x_hbm, idx_hbm, mem_hbm, out_hbm, idx_all, rows_v, out_v, xall_v,
          sem0, sem1):
    wid = lax.axis_index("s") * 2 + lax.axis_index("c")
    b0 = wid * B_PER_W
    iota = lax.iota(jnp.int32, L)
    inv_t = jnp.float32(1.0 / T)

    pltpu.sync_copy(idx_hbm.at[pl.ds(b0 * 4, B_PER_W * 4)], idx_all)
    pltpu.sync_copy(x_hbm.at[pl.ds(b0, B_PER_W)], xall_v)

    _fire(mem_hbm, idx_all, rows_v, 0, 0, sem0)

    def per_chunk(t, _):
        p = t & 1
        a = t >> 1

        @pl.when(p == 0)
        def _():
            @pl.when(t < N_CHUNKS - 1)
            def _():
                _fire(mem_hbm, idx_all, rows_v, t + 1, 1, sem1)
            _wait(mem_hbm, idx_all, rows_v, t, 0, sem0)

        @pl.when(p == 1)
        def _():
            @pl.when(t < N_CHUNKS - 1)
            def _():
                _fire(mem_hbm, idx_all, rows_v, t + 1, 0, sem0)
            _wait(mem_hbm, idx_all, rows_v, t, 1, sem1)

        row_base = p * C_ROWS

        NG = 4

        def per_gquad(g, _):
            row_ids_h = [row_base + g * (NG * L) + h * L + iota
                         for h in range(NG)]
            xvs = [xall_v[a, pl.ds(c * L, L)] for c in range(NC)]

            def rbody(r, carry):
                rot = carry[0]
                accs = list(carry[1:])
                for c in range(NC):
                    cidx = rot + c * L
                    xs = xvs[c].at[rot].get(mode="promise_in_bounds")
                    for h in range(NG):
                        col = plsc.load_gather(rows_v, [row_ids_h[h], cidx])
                        accs[h * NC + c] = accs[h * NC + c] + col * xs
                return ((rot + 1) & (L - 1),) + tuple(accs)

            z = jnp.zeros((L,), jnp.float32)
            res = lax.fori_loop(0, L, rbody, (iota,) + (z,) * (NG * NC),
                                unroll=2)
            accs = res[1:]
            for h in range(NG):
                aa = accs[h * NC:(h + 1) * NC]
                s01 = aa[0] + aa[1]
                s23 = aa[2] + aa[3]
                s45 = aa[4] + aa[5]
                s67 = aa[6] + aa[7]
                out_v[pl.ds(t * C_ROWS + g * NG * L + h * L, L)] = (
                    (s01 + s23) + (s45 + s67)) * inv_t
            return 0

        lax.fori_loop(0, G_PER_C // NG, per_gquad, 0)
        return 0

    lax.fori_loop(0, N_CHUNKS, per_chunk, 0)
    pltpu.sync_copy(out_v, out_hbm.at[pl.ds(b0 * K1, B_PER_W * K1)])


@jax.jit
def _run(x, idx2, memory):
    kfn = pl.kernel(
        _body,
        out_type=jax.ShapeDtypeStruct((B * K1,), jnp.float32),
        mesh=plsc.VectorSubcoreMesh(core_axis_name="c", subcore_axis_name="s"),
        compiler_params=pltpu.CompilerParams(needs_layout_passes=False),
        scratch_types=[
            pltpu.VMEM((B_PER_W * 4, 128), jnp.int32),
            pltpu.VMEM((2 * C_ROWS, D), jnp.float32),
            pltpu.VMEM((B_PER_W * K1,), jnp.float32),
            pltpu.VMEM((B_PER_W, D), jnp.float32),
            pltpu.SemaphoreType.DMA,
            pltpu.SemaphoreType.DMA,
        ],
    )
    return kfn(x, idx2, memory)


def kernel(x, y, idx, memory):
    del y
    idx2 = idx.reshape(B * 4, 128)
    return _run(x, idx2, memory).reshape(B, K1)


# speedup vs baseline: 1.1252x; 1.0072x over previous
import jax
import jax.numpy as jnp
from jax import lax
from jax.experimental import pallas as pl
from jax.experimental.pallas import tpu as pltpu
from jax.experimental.pallas import tpu_sc as plsc

B, D, V, K1 = 1024, 128, 1000000, 512
T = 0.07
L = 16
NW = 32
B_PER_W = B // NW
C_ROWS = 256
N_CHUNKS = B_PER_W * 2
G_PER_C = C_ROWS // L
NC = D // L


def _fire(mem_hbm, idx_all, rows_v, chunk, buf, sem):
    for r in range(2):
        pltpu.async_copy(
            mem_hbm.at[idx_all.at[chunk * 2 + r]],
            rows_v.at[pl.ds(buf * C_ROWS + r * 128, 128)],
            sem,
        )


def _wait(mem_hbm, idx_all, rows_v, chunk, buf, sem):
    for r in range(2):
        pltpu.make_async_copy(
            mem_hbm.at[idx_all.at[chunk * 2 + r]],
            rows_v.at[pl.ds(buf * C_ROWS + r * 128, 128)],
            sem,
        ).wait()


def _body(x_hbm, idx_hbm, mem_hbm, out_hbm, idx_all, rows_v, out_v, xall_v,
          sem0, sem1):
    wid = lax.axis_index("s") * 2 + lax.axis_index("c")
    b0 = wid * B_PER_W
    iota = lax.iota(jnp.int32, L)
    inv_t = jnp.float32(1.0 / T)

    pltpu.sync_copy(idx_hbm.at[pl.ds(b0 * 4, B_PER_W * 4)], idx_all)
    pltpu.sync_copy(x_hbm.at[pl.ds(b0, B_PER_W)], xall_v)

    _fire(mem_hbm, idx_all, rows_v, 0, 0, sem0)

    def per_chunk(t, _):
        p = t & 1
        a = t >> 1

        @pl.when(p == 0)
        def _():
            @pl.when(t < N_CHUNKS - 1)
            def _():
                _fire(mem_hbm, idx_all, rows_v, t + 1, 1, sem1)
            _wait(mem_hbm, idx_all, rows_v, t, 0, sem0)

        @pl.when(p == 1)
        def _():
            @pl.when(t < N_CHUNKS - 1)
            def _():
                _fire(mem_hbm, idx_all, rows_v, t + 1, 0, sem0)
            _wait(mem_hbm, idx_all, rows_v, t, 1, sem1)

        row_base = p * C_ROWS

        NG = 4

        def per_gquad(g, _):
            row_ids_h = [row_base + g * (NG * L) + h * L + iota
                         for h in range(NG)]
            xvs = [xall_v[a, pl.ds(c * L, L)] for c in range(NC)]

            def rbody(r, carry):
                rot = carry[0]
                accs = list(carry[1:])
                for c in range(NC):
                    cidx = rot + c * L
                    xs = xvs[c].at[rot].get(mode="promise_in_bounds")
                    for h in range(NG):
                        col = plsc.load_gather(rows_v, [row_ids_h[h], cidx])
                        accs[h * NC + c] = accs[h * NC + c] + col * xs
                return ((rot + 1) & (L - 1),) + tuple(accs)

            z = jnp.zeros((L,), jnp.float32)
            res = lax.fori_loop(0, L, rbody, (iota,) + (z,) * (NG * NC),
                                unroll=4)
            accs = res[1:]
            for h in range(NG):
                aa = accs[h * NC:(h + 1) * NC]
                s01 = aa[0] + aa[1]
                s23 = aa[2] + aa[3]
                s45 = aa[4] + aa[5]
                s67 = aa[6] + aa[7]
                out_v[pl.ds(t * C_ROWS + g * NG * L + h * L, L)] = (
                    (s01 + s23) + (s45 + s67)) * inv_t
            return 0

        lax.fori_loop(0, G_PER_C // NG, per_gquad, 0)
        return 0

    lax.fori_loop(0, N_CHUNKS, per_chunk, 0)
    pltpu.sync_copy(out_v, out_hbm.at[pl.ds(b0 * K1, B_PER_W * K1)])


@jax.jit
def _run(x, idx2, memory):
    kfn = pl.kernel(
        _body,
        out_type=jax.ShapeDtypeStruct((B * K1,), jnp.float32),
        mesh=plsc.VectorSubcoreMesh(core_axis_name="c", subcore_axis_name="s"),
        compiler_params=pltpu.CompilerParams(needs_layout_passes=False),
        scratch_types=[
            pltpu.VMEM((B_PER_W * 4, 128), jnp.int32),
            pltpu.VMEM((2 * C_ROWS, D), jnp.float32),
            pltpu.VMEM((B_PER_W * K1,), jnp.float32),
            pltpu.VMEM((B_PER_W, D), jnp.float32),
            pltpu.SemaphoreType.DMA,
            pltpu.SemaphoreType.DMA,
        ],
    )
    return kfn(x, idx2, memory)


def kernel(x, y, idx, memory):
    del y
    idx2 = idx.reshape(B * 4, 128)
    return _run(x, idx2, memory).reshape(B, K1)
